# baseline (speedup 1.0000x reference)
import functools

import jax
import jax.numpy as jnp
from jax import lax
from jax.experimental import pallas as pl
from jax.experimental.pallas import tpu as pltpu
from jax.experimental.pallas import tpu_sc as plsc

F32 = jnp.float32

SUB = 128
CHUNK = 2048
NSUB = CHUNK // SUB
D = 16
CORE0_FRAC = 0.145


def _relu(x):
    return jnp.maximum(x, 0.0)


def _ln_rows(x, g, b):
    mu = jnp.mean(x, axis=1, keepdims=True)
    var = jnp.mean((x - mu) ** 2, axis=1, keepdims=True)
    return (x - mu) * lax.rsqrt(var + 1e-5) * g + b


def _pad2(w, r, c):
    return jnp.zeros((r, c), F32).at[: w.shape[0], : w.shape[1]].set(w)


def _pad1(v, n):
    return jnp.zeros((1, n), F32).at[0, : v.shape[0]].set(v)


def _pro_body(vf, cf, ef, lnvg, lnvb, Wv, bv, lncg, lncb, Wc, bc, we, be,
              Wr_v2c, Wo_v2c, br_v2c, Wr_c2v, Wo_c2v, br_c2v,
              vp_o, cp_o, rc_o, rv_o, ew_o):
    v0 = _relu(jnp.dot(_ln_rows(vf[...], lnvg[...], lnvb[...]), Wv[...],
                       preferred_element_type=F32) + bv[...])
    c0 = _relu(jnp.dot(_ln_rows(cf[...], lncg[...], lncb[...]), Wc[...],
                       preferred_element_type=F32) + bc[...])
    vp_o[...] = jnp.dot(v0, Wr_v2c[...], preferred_element_type=F32)
    cp_o[...] = jnp.dot(c0, Wr_c2v[...], preferred_element_type=F32)
    rc_o[...] = jnp.dot(c0, Wo_v2c[...], preferred_element_type=F32) + br_v2c[...]
    rv_o[...] = jnp.dot(v0, Wo_c2v[...], preferred_element_type=F32) + br_c2v[...]
    ew_o[...] = ef[...] * we[0, 0] + be[0, 0]


def _epi_body(accc_lo, accc_hi, accv_lo, accv_hi, rc, rv,
              Wr_v2c, Wo_v2c, br_v2c, Wr_c2v, Wo_c2v, br_c2v,
              vp_o, cp_o, rc_o, rv_o):
    c_new = _relu(accc_lo[...] + accc_hi[...] + rc[...])
    v_new = _relu(accv_lo[...] + accv_hi[...] + rv[...])
    vp_o[...] = jnp.dot(v_new, Wr_v2c[...], preferred_element_type=F32)
    cp_o[...] = jnp.dot(c_new, Wr_c2v[...], preferred_element_type=F32)
    rc_o[...] = jnp.dot(c_new, Wo_v2c[...], preferred_element_type=F32) + br_v2c[...]
    rv_o[...] = jnp.dot(v_new, Wo_c2v[...], preferred_element_type=F32) + br_c2v[...]


def _fin_body(nc, nv, blk, ngrid,
              accc_lo, accc_hi, accv_lo, accv_hi, rc, rv, bb,
              lnbbg, lnbbb, Wbb, bbb, Wp, bp, lnpg, lnpb,
              out, sum_c, sum_v):
    b = pl.program_id(0)
    c3 = _relu(accc_lo[...] + accc_hi[...] + rc[...])
    v3 = _relu(accv_lo[...] + accv_hi[...] + rv[...])
    row = lax.broadcasted_iota(jnp.int32, (blk, D), 0) + b * blk
    pc = jnp.sum(jnp.where(row < nc, c3, 0.0), axis=0, keepdims=True)
    pv = jnp.sum(jnp.where(row < nv, v3, 0.0), axis=0, keepdims=True)

    @pl.when(b == 0)
    def _():
        sum_c[...] = pc
        sum_v[...] = pv

    @pl.when(b > 0)
    def _():
        sum_c[...] = sum_c[...] + pc
        sum_v[...] = sum_v[...] + pv

    @pl.when(b == ngrid - 1)
    def _():
        cm = sum_c[...][:, :4] / nc
        vm = sum_v[...][:, :4] / nv
        bbe = _relu(jnp.dot(_ln_rows(bb[...], lnbbg[...], lnbbb[...]),
                            Wbb[...], preferred_element_type=F32) + bbb[...])
        h = (jnp.dot(vm, Wp[...][0:4, :], preferred_element_type=F32)
             + jnp.dot(cm, Wp[...][4:8, :], preferred_element_type=F32)
             + jnp.dot(bbe, Wp[...][8:10, :], preferred_element_type=F32)
             + bp[...])
        out[...] = _relu(_ln_rows(h, lnpg[...], lnpb[...]))


def _edge_body(R, nch0, nch1,
               ia_h, ib_h, ew_h, vp_h, cp_h, accc_h, accv_h,
               ia_v, ib_v, ew_v, ga, acc, gsem, ssem):
    c = lax.axis_index("c")
    s = lax.axis_index("s")
    trows = R // 16
    base = s * trows
    rem = trows - CHUNK
    obase = c * R + s * trows
    nch = jnp.where(c == 0, nch0, nch1)
    tile0 = (c * 16 + s) * (nch0 * NSUB) + c * s * ((nch1 - nch0) * NSUB)

    def one_pass(tbl_h, gref, sref, out_h):
        def load_idx(kk, buf):
            row0 = tile0 + kk * NSUB
            ioff = buf * NSUB
            pltpu.sync_copy(ia_h.at[pl.ds(row0, NSUB)],
                            ia_v.at[pl.ds(ioff, NSUB)])
            pltpu.sync_copy(ib_h.at[pl.ds(row0, NSUB)],
                            ib_v.at[pl.ds(ioff, NSUB)])
            pltpu.sync_copy(ew_h.at[pl.ds(row0, NSUB)],
                            ew_v.at[pl.ds(ioff, NSUB)])

        def fire_gathers(buf):
            goff = buf * CHUNK

            def gath(j, _):
                pltpu.async_copy(tbl_h.at[gref.at[buf * NSUB + j]],
                                 ga.at[pl.ds(goff + j * SUB, SUB)], gsem)
                return 0
            lax.fori_loop(0, NSUB, gath, 0)

        def drain_gathers(buf):
            pltpu.make_async_copy(tbl_h.at[pl.ds(0, CHUNK)],
                                  ga.at[pl.ds(buf * CHUNK, CHUNK)],
                                  gsem).wait()

        def fire_scatters(buf):
            goff = buf * CHUNK

            def scat(j, _):
                pltpu.async_copy(ga.at[pl.ds(goff + j * SUB, SUB)],
                                 acc.at[sref.at[buf * NSUB + j]], ssem,
                                 add=True)
                return 0
            lax.fori_loop(0, NSUB, scat, 0)

        def drain_scatters(buf):
            pltpu.make_async_copy(ga.at[pl.ds(buf * CHUNK, CHUNK)],
                                  acc.at[pl.ds(0, CHUNK)], ssem).wait()

        def multiply(buf):
            goff = buf * CHUNK

            def mrow_j(j, _):
                def mrow_m(m, _):
                    e16 = ew_v[buf * NSUB + j, pl.ds(m * 16, 16)]
                    base_i = goff + j * SUB + m * 16
                    for t in range(16):
                        ga[base_i + t, :] = ga[base_i + t, :] * e16[t]
                    return 0
                lax.fori_loop(0, SUB // 16, mrow_m, 0)
                return 0
            lax.fori_loop(0, NSUB, mrow_j, 0)

        load_idx(0, 0)
        fire_gathers(0)

        def zrow(i, _):
            ga[CHUNK + i, :] = jnp.zeros((D,), F32)
            return 0
        lax.fori_loop(0, CHUNK, zrow, 0)
        pltpu.sync_copy(ga.at[pl.ds(CHUNK, CHUNK)], acc.at[pl.ds(base, CHUNK)])
        pltpu.sync_copy(ga.at[pl.ds(CHUNK, rem)],
                        acc.at[pl.ds(base + CHUNK, rem)])
        plsc.subcore_barrier()

        def chunk_body(kk, _):
            cur = lax.rem(kk, 2)
            nxt = 1 - cur

            drain_gathers(cur)

            @pl.when(kk + 1 < nch)
            def _():
                load_idx(kk + 1, nxt)

                @pl.when(kk >= 1)
                def _():
                    drain_scatters(nxt)
                fire_gathers(nxt)

            multiply(cur)
            fire_scatters(cur)
            return 0

        lax.fori_loop(0, nch, chunk_body, 0)
        drain_scatters(lax.rem(nch - 2, 2))
        drain_scatters(lax.rem(nch - 1, 2))
        plsc.subcore_barrier()
        pltpu.sync_copy(acc.at[pl.ds(base, trows)], out_h.at[pl.ds(obase, trows)])
        plsc.subcore_barrier()

    one_pass(vp_h, ia_v, ib_v, accc_h)
    one_pass(cp_h, ib_v, ia_v, accv_h)


def _make_edge_pass(R, nch0, nch1):
    mesh = plsc.VectorSubcoreMesh(core_axis_name="c", subcore_axis_name="s")
    return pl.kernel(
        functools.partial(_edge_body, R, nch0, nch1),
        out_type=[jax.ShapeDtypeStruct((2 * R, D), F32),
                  jax.ShapeDtypeStruct((2 * R, D), F32)],
        mesh=mesh,
        compiler_params=pltpu.CompilerParams(use_tc_tiling_on_sc=False),
        scratch_types=[
            pltpu.VMEM((2 * NSUB, SUB), jnp.int32),
            pltpu.VMEM((2 * NSUB, SUB), jnp.int32),
            pltpu.VMEM((2 * NSUB, SUB), F32),
            pltpu.VMEM((2 * CHUNK, D), F32),
            pltpu.VMEM_SHARED((R, D), F32),
            pltpu.SemaphoreType.DMA,
            pltpu.SemaphoreType.DMA,
        ],
    )


def kernel(params, constraint_features, edge_indices, edge_features,
           variable_features, bbounds):
    p = params
    nc = constraint_features.shape[0]
    nv = variable_features.shape[0]
    ne = edge_indices.shape[1]
    R = ((max(nc, nv) + 1 + 127) // 128) * 128
    ntot = (ne + 16 * CHUNK - 1) // (16 * CHUNK)
    nch0 = min(max(2, round(ntot * CORE0_FRAC)), ntot - 2)
    nch1 = ntot - nch0
    epad = 16 * CHUNK * ntot
    blk = R // 16
    ngrid = 16
    erows = epad // SUB
    eblk = erows // ngrid

    vf = jnp.pad(variable_features, ((0, R - nv), (0, 0)))
    cf = jnp.pad(constraint_features, ((0, R - nc), (0, 0)))
    ia = jnp.concatenate(
        [edge_indices[0], jnp.full((epad - ne,), nv, jnp.int32)]).reshape(erows, SUB)
    ib = jnp.concatenate(
        [edge_indices[1], jnp.full((epad - ne,), nc, jnp.int32)]).reshape(erows, SUB)
    ef = jnp.concatenate(
        [edge_features[:, 0], jnp.zeros((epad - ne,), F32)]).reshape(erows, SUB)

    def rnd_w(i):
        r = 32 if i == 0 else D
        return (_pad2(p['v2c_Wrel' + str(i)], r, D),
                _pad2(p['v2c_Wroot' + str(i)], r, D),
                _pad1(p['v2c_brel' + str(i)], D),
                _pad2(p['c2v_Wrel' + str(i)], r, D),
                _pad2(p['c2v_Wroot' + str(i)], r, D),
                _pad1(p['c2v_brel' + str(i)], D))

    full = lambda shape: pl.BlockSpec(shape, lambda b: (0, 0))
    nblk = lambda w: pl.BlockSpec((blk, w), lambda b: (b, 0))

    w0 = rnd_w(0)
    vp, cp, rc, rv, ew = pl.pallas_call(
        _pro_body,
        grid=(ngrid,),
        in_specs=[nblk(10), nblk(6), pl.BlockSpec((eblk, SUB), lambda b: (b, 0)),
                  full((1, 10)), full((1, 10)), full((10, 32)), full((1, 32)),
                  full((1, 6)), full((1, 6)), full((6, 32)), full((1, 32)),
                  full((1, 1)), full((1, 1)),
                  full((32, D)), full((32, D)), full((1, D)),
                  full((32, D)), full((32, D)), full((1, D))],
        out_specs=[nblk(D), nblk(D), nblk(D), nblk(D),
                   pl.BlockSpec((eblk, SUB), lambda b: (b, 0))],
        out_shape=[jax.ShapeDtypeStruct((R, D), F32)] * 4
        + [jax.ShapeDtypeStruct((erows, SUB), F32)],
    )(vf, cf, ef,
      _pad1(p['ln_v_g'], 10), _pad1(p['ln_v_b'], 10), p['W_v'],
      _pad1(p['b_v'], 32),
      _pad1(p['ln_c_g'], 6), _pad1(p['ln_c_b'], 6), p['W_c'],
      _pad1(p['b_c'], 32),
      p['W_e'], p['b_e'].reshape(1, 1),
      *w0)

    edge_pass = _make_edge_pass(R, nch0, nch1)

    lo = lambda: pl.BlockSpec((blk, D), lambda b: (b, 0))
    hi = lambda: pl.BlockSpec((blk, D), lambda b: (b + 16, 0))

    for i in range(3):
        accc, accv = edge_pass(ia, ib, ew, vp, cp)
        if i < 2:
            wi = rnd_w(i + 1)
            vp, cp, rc, rv = pl.pallas_call(
                _epi_body,
                grid=(ngrid,),
                in_specs=[lo(), hi(), lo(), hi(), nblk(D), nblk(D),
                          full((D, D)), full((D, D)), full((1, D)),
                          full((D, D)), full((D, D)), full((1, D))],
                out_specs=[nblk(D)] * 4,
                out_shape=[jax.ShapeDtypeStruct((R, D), F32)] * 4,
            )(accc, accc, accv, accv, rc, rv, *wi)
        else:
            out = pl.pallas_call(
                functools.partial(_fin_body, nc, nv, blk, ngrid),
                grid=(ngrid,),
                in_specs=[lo(), hi(), lo(), hi(), nblk(D), nblk(D),
                          full((1, 2)), full((1, 2)), full((1, 2)),
                          full((2, 2)), full((1, 2)), full((10, 15)),
                          full((1, 15)), full((1, 15)), full((1, 15))],
                out_specs=pl.BlockSpec((1, 15), lambda b: (0, 0)),
                out_shape=jax.ShapeDtypeStruct((1, 15), F32),
                scratch_shapes=[pltpu.VMEM((1, D), F32),
                                pltpu.VMEM((1, D), F32)],
            )(accc, accc, accv, accv, rc, rv, bbounds,
              _pad1(p['ln_bb_g'], 2), _pad1(p['ln_bb_b'], 2),
              p['W_bb'], _pad1(p['b_bb'], 2), p['W_p'],
              _pad1(p['b_p'], 15), _pad1(p['ln_p_g'], 15),
              _pad1(p['ln_p_b'], 15))
    return out


# speedup vs baseline: 1.1051x; 1.1051x over previous
import functools

import jax
import jax.numpy as jnp
from jax import lax
from jax.experimental import pallas as pl
from jax.experimental.pallas import tpu as pltpu
from jax.experimental.pallas import tpu_sc as plsc

F32 = jnp.float32

SUB = 128
CHUNK = 2048
NSUB = CHUNK // SUB
D = 16
CORE0_FRAC = 0.30


def _relu(x):
    return jnp.maximum(x, 0.0)


def _ln_rows(x, g, b):
    mu = jnp.mean(x, axis=1, keepdims=True)
    var = jnp.mean((x - mu) ** 2, axis=1, keepdims=True)
    return (x - mu) * lax.rsqrt(var + 1e-5) * g + b


def _pad2(w, r, c):
    return jnp.zeros((r, c), F32).at[: w.shape[0], : w.shape[1]].set(w)


def _pad1(v, n):
    return jnp.zeros((1, n), F32).at[0, : v.shape[0]].set(v)


def _pro_body(vf, cf, ef, lnvg, lnvb, Wv, bv, lncg, lncb, Wc, bc, we, be,
              Wr_v2c, Wo_v2c, br_v2c, Wr_c2v, Wo_c2v, br_c2v,
              vp_o, cp_o, rc_o, rv_o, ew_o):
    v0 = _relu(jnp.dot(_ln_rows(vf[...], lnvg[...], lnvb[...]), Wv[...],
                       preferred_element_type=F32) + bv[...])
    c0 = _relu(jnp.dot(_ln_rows(cf[...], lncg[...], lncb[...]), Wc[...],
                       preferred_element_type=F32) + bc[...])
    vp_o[...] = jnp.dot(v0, Wr_v2c[...], preferred_element_type=F32)
    cp_o[...] = jnp.dot(c0, Wr_c2v[...], preferred_element_type=F32)
    rc_o[...] = jnp.dot(c0, Wo_v2c[...], preferred_element_type=F32) + br_v2c[...]
    rv_o[...] = jnp.dot(v0, Wo_c2v[...], preferred_element_type=F32) + br_c2v[...]
    ew_o[...] = ef[...] * we[0, 0] + be[0, 0]


def _epi_body(accc_lo, accc_hi, accv_lo, accv_hi, rc, rv,
              Wr_v2c, Wo_v2c, br_v2c, Wr_c2v, Wo_c2v, br_c2v,
              vp_o, cp_o, rc_o, rv_o):
    c_new = _relu(accc_lo[...] + accc_hi[...] + rc[...])
    v_new = _relu(accv_lo[...] + accv_hi[...] + rv[...])
    vp_o[...] = jnp.dot(v_new, Wr_v2c[...], preferred_element_type=F32)
    cp_o[...] = jnp.dot(c_new, Wr_c2v[...], preferred_element_type=F32)
    rc_o[...] = jnp.dot(c_new, Wo_v2c[...], preferred_element_type=F32) + br_v2c[...]
    rv_o[...] = jnp.dot(v_new, Wo_c2v[...], preferred_element_type=F32) + br_c2v[...]


def _fin_body(nc, nv, blk, ngrid,
              accc_lo, accc_hi, accv_lo, accv_hi, rc, rv, bb,
              lnbbg, lnbbb, Wbb, bbb, Wp, bp, lnpg, lnpb,
              out, sum_c, sum_v):
    b = pl.program_id(0)
    c3 = _relu(accc_lo[...] + accc_hi[...] + rc[...])
    v3 = _relu(accv_lo[...] + accv_hi[...] + rv[...])
    row = lax.broadcasted_iota(jnp.int32, (blk, D), 0) + b * blk
    pc = jnp.sum(jnp.where(row < nc, c3, 0.0), axis=0, keepdims=True)
    pv = jnp.sum(jnp.where(row < nv, v3, 0.0), axis=0, keepdims=True)

    @pl.when(b == 0)
    def _():
        sum_c[...] = pc
        sum_v[...] = pv

    @pl.when(b > 0)
    def _():
        sum_c[...] = sum_c[...] + pc
        sum_v[...] = sum_v[...] + pv

    @pl.when(b == ngrid - 1)
    def _():
        cm = sum_c[...][:, :4] / nc
        vm = sum_v[...][:, :4] / nv
        bbe = _relu(jnp.dot(_ln_rows(bb[...], lnbbg[...], lnbbb[...]),
                            Wbb[...], preferred_element_type=F32) + bbb[...])
        h = (jnp.dot(vm, Wp[...][0:4, :], preferred_element_type=F32)
             + jnp.dot(cm, Wp[...][4:8, :], preferred_element_type=F32)
             + jnp.dot(bbe, Wp[...][8:10, :], preferred_element_type=F32)
             + bp[...])
        out[...] = _relu(_ln_rows(h, lnpg[...], lnpb[...]))


def _edge_body(R, nch0, nch1,
               ia_h, ib_h, ew_h, vp_h, cp_h, accc_h, accv_h,
               ia_v, ib_v, ew_v, ga, acc, gsem, ssem):
    c = lax.axis_index("c")
    s = lax.axis_index("s")
    trows = R // 16
    base = s * trows
    rem = trows - CHUNK
    obase = c * R + s * trows
    nch = jnp.where(c == 0, nch0, nch1)
    tile0 = (c * 16 + s) * (nch0 * NSUB) + c * s * ((nch1 - nch0) * NSUB)

    def one_pass(tbl_h, gref, sref, out_h):
        def load_idx(kk, buf):
            row0 = tile0 + kk * NSUB
            ioff = buf * NSUB
            pltpu.sync_copy(ia_h.at[pl.ds(row0, NSUB)],
                            ia_v.at[pl.ds(ioff, NSUB)])
            pltpu.sync_copy(ib_h.at[pl.ds(row0, NSUB)],
                            ib_v.at[pl.ds(ioff, NSUB)])
            pltpu.sync_copy(ew_h.at[pl.ds(row0, NSUB)],
                            ew_v.at[pl.ds(ioff, NSUB)])

        def fire_gathers(buf):
            goff = buf * CHUNK

            def gath(j, _):
                pltpu.async_copy(tbl_h.at[gref.at[buf * NSUB + j]],
                                 ga.at[pl.ds(goff + j * SUB, SUB)], gsem)
                return 0
            lax.fori_loop(0, NSUB, gath, 0)

        def drain_gathers(buf):
            pltpu.make_async_copy(tbl_h.at[pl.ds(0, CHUNK)],
                                  ga.at[pl.ds(buf * CHUNK, CHUNK)],
                                  gsem).wait()

        def fire_scatters(buf):
            goff = buf * CHUNK

            def scat(j, _):
                pltpu.async_copy(ga.at[pl.ds(goff + j * SUB, SUB)],
                                 acc.at[sref.at[buf * NSUB + j]], ssem,
                                 add=True)
                return 0
            lax.fori_loop(0, NSUB, scat, 0)

        def drain_scatters(buf):
            pltpu.make_async_copy(ga.at[pl.ds(buf * CHUNK, CHUNK)],
                                  acc.at[pl.ds(0, CHUNK)], ssem).wait()

        def multiply(buf):
            goff = buf * CHUNK

            @plsc.parallel_loop(0, CHUNK // 16, unroll=2)
            def _(m):
                e16 = ew_v[buf * NSUB + m // (SUB // 16),
                           pl.ds((m % (SUB // 16)) * 16, 16)]
                base_i = goff + m * 16
                for t in range(16):
                    ga[base_i + t, :] = ga[base_i + t, :] * e16[t]

        load_idx(0, 0)
        fire_gathers(0)

        def zrow(i, _):
            ga[CHUNK + i, :] = jnp.zeros((D,), F32)
            return 0
        lax.fori_loop(0, CHUNK, zrow, 0)
        pltpu.sync_copy(ga.at[pl.ds(CHUNK, CHUNK)], acc.at[pl.ds(base, CHUNK)])
        pltpu.sync_copy(ga.at[pl.ds(CHUNK, rem)],
                        acc.at[pl.ds(base + CHUNK, rem)])
        plsc.subcore_barrier()

        def chunk_body(kk, _):
            cur = lax.rem(kk, 2)
            nxt = 1 - cur

            drain_gathers(cur)

            @pl.when(kk + 1 < nch)
            def _():
                load_idx(kk + 1, nxt)

                @pl.when(kk >= 1)
                def _():
                    drain_scatters(nxt)
                fire_gathers(nxt)

            multiply(cur)
            fire_scatters(cur)
            return 0

        lax.fori_loop(0, nch, chunk_body, 0)
        drain_scatters(lax.rem(nch - 2, 2))
        drain_scatters(lax.rem(nch - 1, 2))
        plsc.subcore_barrier()
        pltpu.sync_copy(acc.at[pl.ds(base, trows)], out_h.at[pl.ds(obase, trows)])
        plsc.subcore_barrier()

    one_pass(vp_h, ia_v, ib_v, accc_h)
    one_pass(cp_h, ib_v, ia_v, accv_h)


def _make_edge_pass(R, nch0, nch1):
    mesh = plsc.VectorSubcoreMesh(core_axis_name="c", subcore_axis_name="s")
    return pl.kernel(
        functools.partial(_edge_body, R, nch0, nch1),
        out_type=[jax.ShapeDtypeStruct((2 * R, D), F32),
                  jax.ShapeDtypeStruct((2 * R, D), F32)],
        mesh=mesh,
        compiler_params=pltpu.CompilerParams(use_tc_tiling_on_sc=False),
        scratch_types=[
            pltpu.VMEM((2 * NSUB, SUB), jnp.int32),
            pltpu.VMEM((2 * NSUB, SUB), jnp.int32),
            pltpu.VMEM((2 * NSUB, SUB), F32),
            pltpu.VMEM((2 * CHUNK, D), F32),
            pltpu.VMEM_SHARED((R, D), F32),
            pltpu.SemaphoreType.DMA,
            pltpu.SemaphoreType.DMA,
        ],
    )


def kernel(params, constraint_features, edge_indices, edge_features,
           variable_features, bbounds):
    p = params
    nc = constraint_features.shape[0]
    nv = variable_features.shape[0]
    ne = edge_indices.shape[1]
    R = ((max(nc, nv) + 1 + 127) // 128) * 128
    ntot = (ne + 16 * CHUNK - 1) // (16 * CHUNK)
    nch0 = min(max(2, round(ntot * CORE0_FRAC)), ntot - 2)
    nch1 = ntot - nch0
    epad = 16 * CHUNK * ntot
    blk = R // 16
    ngrid = 16
    erows = epad // SUB
    eblk = erows // ngrid

    vf = jnp.pad(variable_features, ((0, R - nv), (0, 0)))
    cf = jnp.pad(constraint_features, ((0, R - nc), (0, 0)))
    ia = jnp.concatenate(
        [edge_indices[0], jnp.full((epad - ne,), nv, jnp.int32)]).reshape(erows, SUB)
    ib = jnp.concatenate(
        [edge_indices[1], jnp.full((epad - ne,), nc, jnp.int32)]).reshape(erows, SUB)
    ef = jnp.concatenate(
        [edge_features[:, 0], jnp.zeros((epad - ne,), F32)]).reshape(erows, SUB)

    def rnd_w(i):
        r = 32 if i == 0 else D
        return (_pad2(p['v2c_Wrel' + str(i)], r, D),
                _pad2(p['v2c_Wroot' + str(i)], r, D),
                _pad1(p['v2c_brel' + str(i)], D),
                _pad2(p['c2v_Wrel' + str(i)], r, D),
                _pad2(p['c2v_Wroot' + str(i)], r, D),
                _pad1(p['c2v_brel' + str(i)], D))

    full = lambda shape: pl.BlockSpec(shape, lambda b: (0, 0))
    nblk = lambda w: pl.BlockSpec((blk, w), lambda b: (b, 0))

    w0 = rnd_w(0)
    vp, cp, rc, rv, ew = pl.pallas_call(
        _pro_body,
        grid=(ngrid,),
        in_specs=[nblk(10), nblk(6), pl.BlockSpec((eblk, SUB), lambda b: (b, 0)),
                  full((1, 10)), full((1, 10)), full((10, 32)), full((1, 32)),
                  full((1, 6)), full((1, 6)), full((6, 32)), full((1, 32)),
                  full((1, 1)), full((1, 1)),
                  full((32, D)), full((32, D)), full((1, D)),
                  full((32, D)), full((32, D)), full((1, D))],
        out_specs=[nblk(D), nblk(D), nblk(D), nblk(D),
                   pl.BlockSpec((eblk, SUB), lambda b: (b, 0))],
        out_shape=[jax.ShapeDtypeStruct((R, D), F32)] * 4
        + [jax.ShapeDtypeStruct((erows, SUB), F32)],
    )(vf, cf, ef,
      _pad1(p['ln_v_g'], 10), _pad1(p['ln_v_b'], 10), p['W_v'],
      _pad1(p['b_v'], 32),
      _pad1(p['ln_c_g'], 6), _pad1(p['ln_c_b'], 6), p['W_c'],
      _pad1(p['b_c'], 32),
      p['W_e'], p['b_e'].reshape(1, 1),
      *w0)

    edge_pass = _make_edge_pass(R, nch0, nch1)

    lo = lambda: pl.BlockSpec((blk, D), lambda b: (b, 0))
    hi = lambda: pl.BlockSpec((blk, D), lambda b: (b + 16, 0))

    for i in range(3):
        accc, accv = edge_pass(ia, ib, ew, vp, cp)
        if i < 2:
            wi = rnd_w(i + 1)
            vp, cp, rc, rv = pl.pallas_call(
                _epi_body,
                grid=(ngrid,),
                in_specs=[lo(), hi(), lo(), hi(), nblk(D), nblk(D),
                          full((D, D)), full((D, D)), full((1, D)),
                          full((D, D)), full((D, D)), full((1, D))],
                out_specs=[nblk(D)] * 4,
                out_shape=[jax.ShapeDtypeStruct((R, D), F32)] * 4,
            )(accc, accc, accv, accv, rc, rv, *wi)
        else:
            out = pl.pallas_call(
                functools.partial(_fin_body, nc, nv, blk, ngrid),
                grid=(ngrid,),
                in_specs=[lo(), hi(), lo(), hi(), nblk(D), nblk(D),
                          full((1, 2)), full((1, 2)), full((1, 2)),
                          full((2, 2)), full((1, 2)), full((10, 15)),
                          full((1, 15)), full((1, 15)), full((1, 15))],
                out_specs=pl.BlockSpec((1, 15), lambda b: (0, 0)),
                out_shape=jax.ShapeDtypeStruct((1, 15), F32),
                scratch_shapes=[pltpu.VMEM((1, D), F32),
                                pltpu.VMEM((1, D), F32)],
            )(accc, accc, accv, accv, rc, rv, bbounds,
              _pad1(p['ln_bb_g'], 2), _pad1(p['ln_bb_b'], 2),
              p['W_bb'], _pad1(p['b_bb'], 2), p['W_p'],
              _pad1(p['b_p'], 15), _pad1(p['ln_p_g'], 15),
              _pad1(p['ln_p_b'], 15))
    return out
